# baseline (speedup 1.0000x reference)
import functools

import jax
import jax.numpy as jnp
from jax import lax
from jax.experimental import pallas as pl
from jax.experimental.pallas import tpu as pltpu
from jax.experimental.pallas import tpu_sc as plsc

N = 10000
C = 128
NPAD = 10240
E = 320000

NCORES = 2
NSUB = 16
NTILES = NCORES * NSUB
CHUNK = 128
CHUNKS_PER_TILE = 80
NCHUNK = NTILES * CHUNKS_PER_TILE
E_PAD = NCHUNK * CHUNK
ROWS_PER_TILE = NPAD // NSUB

RB = 512
GRID = NPAD // RB

K_CORE0 = 158

@functools.cache
def _mesh():
    return plsc.VectorSubcoreMesh(core_axis_name="c", subcore_axis_name="s")


def _sc_degree_body(dst_hbm, zeros_hbm, ones_hbm, out_hbm, dst_v0, dst_v1,
                    ones_v, acc_sh, dsem0, dsem1):
    c = lax.axis_index("c")
    s = lax.axis_index("s")
    base = (c * NSUB + s) * CHUNKS_PER_TILE
    dst0, dst1 = dst_v0, dst_v1
    sem0, sem1 = dsem0, dsem1
    pltpu.sync_copy(zeros_hbm, acc_sh.at[pl.ds(s * ROWS_PER_TILE, ROWS_PER_TILE)])
    pltpu.sync_copy(ones_hbm, ones_v)
    plsc.subcore_barrier()

    def start(j, dstb, semb):
        pltpu.async_copy(dst_hbm.at[base + j], dstb, semb)

    def finish(j, dstb, semb):
        pltpu.make_async_copy(dst_hbm.at[base + j], dstb, semb).wait()
        pltpu.sync_copy(ones_v, acc_sh.at[dstb], add=True)

    start(0, dst0, sem0)
    start(1, dst1, sem1)

    @pl.loop(0, CHUNKS_PER_TILE // 2 - 1)
    def _(t):
        j = 2 * t
        finish(j, dst0, sem0)
        start(j + 2, dst0, sem0)
        finish(j + 1, dst1, sem1)
        start(j + 3, dst1, sem1)

    finish(CHUNKS_PER_TILE - 2, dst0, sem0)
    finish(CHUNKS_PER_TILE - 1, dst1, sem1)

    plsc.subcore_barrier()
    pltpu.sync_copy(acc_sh.at[pl.ds(s * ROWS_PER_TILE, ROWS_PER_TILE)],
                    out_hbm.at[pl.ds(c * NPAD + s * ROWS_PER_TILE,
                                     ROWS_PER_TILE)])


@jax.jit
def _sc_degree(dstp, zeros128, ones128):
    f = pl.kernel(
        _sc_degree_body,
        out_type=jax.ShapeDtypeStruct((NCORES * NPAD, C), jnp.float32),
        mesh=_mesh(),
        scratch_types=[
            pltpu.VMEM((CHUNK,), jnp.int32),
            pltpu.VMEM((CHUNK,), jnp.int32),
            pltpu.VMEM((CHUNK, C), jnp.float32),
            pltpu.VMEM_SHARED((NPAD, C), jnp.float32),
            pltpu.SemaphoreType.DMA,
            pltpu.SemaphoreType.DMA,
        ],
    )
    return f(dstp, zeros128, ones128)


def _sc_prop_body(z_hbm, src_hbm, dst_hbm, zeros_hbm, out_hbm,
                  src0, dst0, rows0, src1, dst1, rows1, acc_sh, sem0, sem1):
    c = lax.axis_index("c")
    s = lax.axis_index("s")
    k = jnp.where(c == 0, K_CORE0, NCHUNK // NSUB - K_CORE0)
    base = jnp.where(c == 0, s * K_CORE0,
                     NSUB * K_CORE0 + s * (NCHUNK // NSUB - K_CORE0))

    def start(j, srcb, dstb, rowsb, sem):
        pltpu.sync_copy(src_hbm.at[base + j], srcb)
        pltpu.sync_copy(dst_hbm.at[base + j], dstb)
        pltpu.async_copy(z_hbm.at[srcb], rowsb, sem)

    def finish(srcb, dstb, rowsb, sem):
        pltpu.make_async_copy(z_hbm.at[srcb], rowsb, sem).wait()
        pltpu.sync_copy(rowsb, acc_sh.at[dstb], add=True)

    pltpu.sync_copy(zeros_hbm, acc_sh.at[pl.ds(s * ROWS_PER_TILE, ROWS_PER_TILE)])
    plsc.subcore_barrier()

    start(0, src0, dst0, rows0, sem0)
    start(1, src1, dst1, rows1, sem1)

    @pl.loop(0, k // 2 - 1)
    def _(t):
        j = 2 * t
        finish(src0, dst0, rows0, sem0)
        start(j + 2, src0, dst0, rows0, sem0)
        finish(src1, dst1, rows1, sem1)
        start(j + 3, src1, dst1, rows1, sem1)

    finish(src0, dst0, rows0, sem0)
    finish(src1, dst1, rows1, sem1)

    plsc.subcore_barrier()
    pltpu.sync_copy(acc_sh.at[pl.ds(s * ROWS_PER_TILE, ROWS_PER_TILE)],
                    out_hbm.at[pl.ds(c * NPAD + s * ROWS_PER_TILE,
                                     ROWS_PER_TILE)])


@jax.jit
def _sc_prop(z, srcp, dstp, zrows):
    f = pl.kernel(
        _sc_prop_body,
        out_type=jax.ShapeDtypeStruct((NCORES * NPAD, C), jnp.float32),
        mesh=_mesh(),
        scratch_types=[
            pltpu.VMEM((CHUNK,), jnp.int32),
            pltpu.VMEM((CHUNK,), jnp.int32),
            pltpu.VMEM((CHUNK, C), jnp.float32),
            pltpu.VMEM((CHUNK,), jnp.int32),
            pltpu.VMEM((CHUNK,), jnp.int32),
            pltpu.VMEM((CHUNK, C), jnp.float32),
            pltpu.VMEM_SHARED((NPAD, C), jnp.float32),
            pltpu.SemaphoreType.DMA,
            pltpu.SemaphoreType.DMA,
        ],
    )
    return f(z, srcp, dstp, zrows)


def _eid_body(emb_ref, w_ref, b_ref, out_ref):
    out_ref[...] = (
        jnp.dot(emb_ref[...], w_ref[...], preferred_element_type=jnp.float32)
        + b_ref[...]
    )


@jax.jit
def _tc_eid(emb_p, w_p, b):
    return pl.pallas_call(
        _eid_body,
        out_shape=jax.ShapeDtypeStruct((16, C), jnp.float32),
    )(emb_p, w_p, b)


def _tc1_body(deg_ref, x_ref, w_ref, z_ref, dinv_ref):
    deg = (jnp.sum(deg_ref[0], axis=1) + jnp.sum(deg_ref[1], axis=1)) * (1.0 / C) + 1.0
    dinv = lax.rsqrt(deg)[:, None]
    y = jnp.dot(x_ref[...], w_ref[...], preferred_element_type=jnp.float32)
    z_ref[...] = y * dinv
    dinv_ref[...] = jnp.broadcast_to(dinv, (RB, C))


@jax.jit
def _tc1(degp, x, w1):
    return pl.pallas_call(
        _tc1_body,
        grid=(GRID,),
        in_specs=[
            pl.BlockSpec((NCORES, RB, C), lambda i: (0, i, 0)),
            pl.BlockSpec((RB, C), lambda i: (i, 0)),
            pl.BlockSpec((C, C), lambda i: (0, 0)),
        ],
        out_specs=[
            pl.BlockSpec((RB, C), lambda i: (i, 0)),
            pl.BlockSpec((RB, C), lambda i: (i, 0)),
        ],
        out_shape=[
            jax.ShapeDtypeStruct((NPAD, C), jnp.float32),
            jax.ShapeDtypeStruct((NPAD, C), jnp.float32),
        ],
    )(degp, x, w1)


def _tc2_body(pa_ref, pb_ref, z1_ref, dinv_ref, b1_ref, w2_ref, z2_ref):
    dinv = dinv_ref[...]
    h = dinv * (pa_ref[...] + pb_ref[...] + z1_ref[...]) + b1_ref[...]
    h = jnp.maximum(h, 0.0)
    z2_ref[...] = (
        jnp.dot(h, w2_ref[...], preferred_element_type=jnp.float32) * dinv
    )


@jax.jit
def _tc2(pa, pb, z1, dinv, b1, w2):
    blk = pl.BlockSpec((RB, C), lambda i: (i, 0))
    return pl.pallas_call(
        _tc2_body,
        grid=(GRID,),
        in_specs=[blk, blk, blk, blk,
                  pl.BlockSpec((1, C), lambda i: (0, 0)),
                  pl.BlockSpec((C, C), lambda i: (0, 0))],
        out_specs=blk,
        out_shape=jax.ShapeDtypeStruct((NPAD, C), jnp.float32),
    )(pa, pb, z1, dinv, b1, w2)


def _tc3_body(pa_ref, pb_ref, z2_ref, dinv_ref, b2_ref, out_ref):
    out_ref[...] = (
        dinv_ref[...] * (pa_ref[...] + pb_ref[...] + z2_ref[...]) + b2_ref[...]
    )


@jax.jit
def _tc3(pa, pb, z2, dinv, b2):
    blk = pl.BlockSpec((RB, C), lambda i: (i, 0))
    return pl.pallas_call(
        _tc3_body,
        grid=(GRID,),
        in_specs=[blk, blk, blk, blk, pl.BlockSpec((1, C), lambda i: (0, 0))],
        out_specs=blk,
        out_shape=jax.ShapeDtypeStruct((NPAD, C), jnp.float32),
    )(pa, pb, z2, dinv, b2)


def kernel(edge_index, init_emb, W_map, b_map, s_emb, W1, b1, W2, b2):
    src = edge_index[0].astype(jnp.int32)
    dst = edge_index[1].astype(jnp.int32)
    pad = jnp.full((E_PAD - E,), N, jnp.int32)
    srcp = jnp.concatenate([src, pad]).reshape(NCHUNK, CHUNK)
    dstp = jnp.concatenate([dst, pad]).reshape(NCHUNK, CHUNK)

    ones128 = jnp.ones((CHUNK, C), jnp.float32)
    zrows = jnp.zeros((ROWS_PER_TILE, C), jnp.float32)

    degp = _sc_degree(dstp, zrows, ones128).reshape(NCORES, NPAD, C)

    emb_p = jnp.zeros((16, C), jnp.float32).at[:, : W_map.shape[0]].set(init_emb)
    wmap_p = jnp.zeros((C, C), jnp.float32).at[: W_map.shape[0]].set(W_map)
    eid_x = _tc_eid(emb_p, wmap_p, b_map.reshape(1, C))
    x = jnp.concatenate(
        [eid_x, s_emb, jnp.zeros((NPAD - N, C), jnp.float32)], axis=0
    )

    z1, dinv = _tc1(degp, x, W1)
    p1 = _sc_prop(z1, srcp, dstp, zrows).reshape(NCORES, NPAD, C)
    z2 = _tc2(p1[0], p1[1], z1, dinv, b1.reshape(1, C), W2)
    p2 = _sc_prop(z2, srcp, dstp, zrows).reshape(NCORES, NPAD, C)
    out = _tc3(p2[0], p2[1], z2, dinv, b2.reshape(1, C))
    return out[:N]


# speedup vs baseline: 1.1932x; 1.1932x over previous
import functools

import jax
import jax.numpy as jnp
from jax import lax
from jax.experimental import pallas as pl
from jax.experimental.pallas import tpu as pltpu
from jax.experimental.pallas import tpu_sc as plsc

N = 10000
C = 128
NPAD = 10240
E = 320000

NCORES = 2
NSUB = 16
NTILES = NCORES * NSUB
CHUNK = 128
CHUNKS_PER_TILE = 80
NCHUNK = NTILES * CHUNKS_PER_TILE
E_PAD = NCHUNK * CHUNK
ROWS_PER_TILE = NPAD // NSUB

RB = 512
GRID = NPAD // RB

K_CORE0 = 140

@functools.cache
def _mesh():
    return plsc.VectorSubcoreMesh(core_axis_name="c", subcore_axis_name="s")


def _sc_degree_body(dst_hbm, zeros_hbm, ones_hbm, out_hbm, dst_v0, dst_v1,
                    ones_v, acc_sh, dsem0, dsem1):
    c = lax.axis_index("c")
    s = lax.axis_index("s")
    base = (c * NSUB + s) * CHUNKS_PER_TILE
    dst0, dst1 = dst_v0, dst_v1
    sem0, sem1 = dsem0, dsem1
    pltpu.sync_copy(zeros_hbm, acc_sh.at[pl.ds(s * ROWS_PER_TILE, ROWS_PER_TILE)])
    pltpu.sync_copy(ones_hbm, ones_v)
    plsc.subcore_barrier()

    def start(j, dstb, semb):
        pltpu.async_copy(dst_hbm.at[base + j], dstb, semb)

    def finish(j, dstb, semb):
        pltpu.make_async_copy(dst_hbm.at[base + j], dstb, semb).wait()
        pltpu.sync_copy(ones_v, acc_sh.at[dstb], add=True)

    start(0, dst0, sem0)
    start(1, dst1, sem1)

    @pl.loop(0, CHUNKS_PER_TILE // 2 - 1)
    def _(t):
        j = 2 * t
        finish(j, dst0, sem0)
        start(j + 2, dst0, sem0)
        finish(j + 1, dst1, sem1)
        start(j + 3, dst1, sem1)

    finish(CHUNKS_PER_TILE - 2, dst0, sem0)
    finish(CHUNKS_PER_TILE - 1, dst1, sem1)

    plsc.subcore_barrier()
    pltpu.sync_copy(acc_sh.at[pl.ds(s * ROWS_PER_TILE, ROWS_PER_TILE)],
                    out_hbm.at[pl.ds(c * NPAD + s * ROWS_PER_TILE,
                                     ROWS_PER_TILE)])


@jax.jit
def _sc_degree(dstp, zeros128, ones128):
    f = pl.kernel(
        _sc_degree_body,
        out_type=jax.ShapeDtypeStruct((NCORES * NPAD, C), jnp.float32),
        mesh=_mesh(),
        scratch_types=[
            pltpu.VMEM((CHUNK,), jnp.int32),
            pltpu.VMEM((CHUNK,), jnp.int32),
            pltpu.VMEM((CHUNK, C), jnp.float32),
            pltpu.VMEM_SHARED((NPAD, C), jnp.float32),
            pltpu.SemaphoreType.DMA,
            pltpu.SemaphoreType.DMA,
        ],
    )
    return f(dstp, zeros128, ones128)


def _sc_prop_body(z_hbm, src_hbm, dst_hbm, zeros_hbm, out_hbm,
                  src0, dst0, rows0, src1, dst1, rows1, acc_sh, sem0, sem1):
    c = lax.axis_index("c")
    s = lax.axis_index("s")
    k = jnp.where(c == 0, K_CORE0, NCHUNK // NSUB - K_CORE0)
    base = jnp.where(c == 0, s * K_CORE0,
                     NSUB * K_CORE0 + s * (NCHUNK // NSUB - K_CORE0))

    def start(j, srcb, dstb, rowsb, sem):
        pltpu.sync_copy(src_hbm.at[base + j], srcb)
        pltpu.sync_copy(dst_hbm.at[base + j], dstb)
        pltpu.async_copy(z_hbm.at[srcb], rowsb, sem)

    def finish(srcb, dstb, rowsb, sem):
        pltpu.make_async_copy(z_hbm.at[srcb], rowsb, sem).wait()
        pltpu.sync_copy(rowsb, acc_sh.at[dstb], add=True)

    pltpu.sync_copy(zeros_hbm, acc_sh.at[pl.ds(s * ROWS_PER_TILE, ROWS_PER_TILE)])
    plsc.subcore_barrier()

    start(0, src0, dst0, rows0, sem0)
    start(1, src1, dst1, rows1, sem1)

    @pl.loop(0, k // 2 - 1)
    def _(t):
        j = 2 * t
        finish(src0, dst0, rows0, sem0)
        start(j + 2, src0, dst0, rows0, sem0)
        finish(src1, dst1, rows1, sem1)
        start(j + 3, src1, dst1, rows1, sem1)

    finish(src0, dst0, rows0, sem0)
    finish(src1, dst1, rows1, sem1)

    plsc.subcore_barrier()
    pltpu.sync_copy(acc_sh.at[pl.ds(s * ROWS_PER_TILE, ROWS_PER_TILE)],
                    out_hbm.at[pl.ds(c * NPAD + s * ROWS_PER_TILE,
                                     ROWS_PER_TILE)])


@jax.jit
def _sc_prop(z, srcp, dstp, zrows):
    f = pl.kernel(
        _sc_prop_body,
        out_type=jax.ShapeDtypeStruct((NCORES * NPAD, C), jnp.float32),
        mesh=_mesh(),
        scratch_types=[
            pltpu.VMEM((CHUNK,), jnp.int32),
            pltpu.VMEM((CHUNK,), jnp.int32),
            pltpu.VMEM((CHUNK, C), jnp.float32),
            pltpu.VMEM((CHUNK,), jnp.int32),
            pltpu.VMEM((CHUNK,), jnp.int32),
            pltpu.VMEM((CHUNK, C), jnp.float32),
            pltpu.VMEM_SHARED((NPAD, C), jnp.float32),
            pltpu.SemaphoreType.DMA,
            pltpu.SemaphoreType.DMA,
        ],
    )
    return f(z, srcp, dstp, zrows)


def _eid_body(emb_ref, w_ref, b_ref, out_ref):
    out_ref[...] = (
        jnp.dot(emb_ref[...], w_ref[...], preferred_element_type=jnp.float32)
        + b_ref[...]
    )


@jax.jit
def _tc_eid(emb_p, w_p, b):
    return pl.pallas_call(
        _eid_body,
        out_shape=jax.ShapeDtypeStruct((16, C), jnp.float32),
    )(emb_p, w_p, b)


def _tc1_body(deg_ref, x_ref, w_ref, z_ref, dinv_ref):
    deg = (jnp.sum(deg_ref[0], axis=1) + jnp.sum(deg_ref[1], axis=1)) * (1.0 / C) + 1.0
    dinv = lax.rsqrt(deg)[:, None]
    y = jnp.dot(x_ref[...], w_ref[...], preferred_element_type=jnp.float32)
    z_ref[...] = y * dinv
    dinv_ref[...] = jnp.broadcast_to(dinv, (RB, C))


@jax.jit
def _tc1(degp, x, w1):
    return pl.pallas_call(
        _tc1_body,
        grid=(GRID,),
        in_specs=[
            pl.BlockSpec((NCORES, RB, C), lambda i: (0, i, 0)),
            pl.BlockSpec((RB, C), lambda i: (i, 0)),
            pl.BlockSpec((C, C), lambda i: (0, 0)),
        ],
        out_specs=[
            pl.BlockSpec((RB, C), lambda i: (i, 0)),
            pl.BlockSpec((RB, C), lambda i: (i, 0)),
        ],
        out_shape=[
            jax.ShapeDtypeStruct((NPAD, C), jnp.float32),
            jax.ShapeDtypeStruct((NPAD, C), jnp.float32),
        ],
    )(degp, x, w1)


def _tc2_body(pa_ref, pb_ref, z1_ref, dinv_ref, b1_ref, w2_ref, z2_ref):
    dinv = dinv_ref[...]
    h = dinv * (pa_ref[...] + pb_ref[...] + z1_ref[...]) + b1_ref[...]
    h = jnp.maximum(h, 0.0)
    z2_ref[...] = (
        jnp.dot(h, w2_ref[...], preferred_element_type=jnp.float32) * dinv
    )


@jax.jit
def _tc2(pa, pb, z1, dinv, b1, w2):
    blk = pl.BlockSpec((RB, C), lambda i: (i, 0))
    return pl.pallas_call(
        _tc2_body,
        grid=(GRID,),
        in_specs=[blk, blk, blk, blk,
                  pl.BlockSpec((1, C), lambda i: (0, 0)),
                  pl.BlockSpec((C, C), lambda i: (0, 0))],
        out_specs=blk,
        out_shape=jax.ShapeDtypeStruct((NPAD, C), jnp.float32),
    )(pa, pb, z1, dinv, b1, w2)


def _tc3_body(pa_ref, pb_ref, z2_ref, dinv_ref, b2_ref, out_ref):
    out_ref[...] = (
        dinv_ref[...] * (pa_ref[...] + pb_ref[...] + z2_ref[...]) + b2_ref[...]
    )


@jax.jit
def _tc3(pa, pb, z2, dinv, b2):
    blk = pl.BlockSpec((RB, C), lambda i: (i, 0))
    return pl.pallas_call(
        _tc3_body,
        grid=(GRID,),
        in_specs=[blk, blk, blk, blk, pl.BlockSpec((1, C), lambda i: (0, 0))],
        out_specs=blk,
        out_shape=jax.ShapeDtypeStruct((NPAD, C), jnp.float32),
    )(pa, pb, z2, dinv, b2)


def kernel(edge_index, init_emb, W_map, b_map, s_emb, W1, b1, W2, b2):
    src = edge_index[0].astype(jnp.int32)
    dst = edge_index[1].astype(jnp.int32)
    pad = jnp.full((E_PAD - E,), N, jnp.int32)
    srcp = jnp.concatenate([src, pad]).reshape(NCHUNK, CHUNK)
    dstp = jnp.concatenate([dst, pad]).reshape(NCHUNK, CHUNK)

    ones128 = jnp.ones((CHUNK, C), jnp.float32)
    zrows = jnp.zeros((ROWS_PER_TILE, C), jnp.float32)

    degp = _sc_degree(dstp, zrows, ones128).reshape(NCORES, NPAD, C)

    emb_p = jnp.zeros((16, C), jnp.float32).at[:, : W_map.shape[0]].set(init_emb)
    wmap_p = jnp.zeros((C, C), jnp.float32).at[: W_map.shape[0]].set(W_map)
    eid_x = _tc_eid(emb_p, wmap_p, b_map.reshape(1, C))
    x = jnp.concatenate(
        [eid_x, s_emb, jnp.zeros((NPAD - N, C), jnp.float32)], axis=0
    )

    z1, dinv = _tc1(degp, x, W1)
    p1 = _sc_prop(z1, srcp, dstp, zrows).reshape(NCORES, NPAD, C)
    z2 = _tc2(p1[0], p1[1], z1, dinv, b1.reshape(1, C), W2)
    p2 = _sc_prop(z2, srcp, dstp, zrows).reshape(NCORES, NPAD, C)
    out = _tc3(p2[0], p2[1], z2, dinv, b2.reshape(1, C))
    return out[:N]
